# baseline (speedup 1.0000x reference)
import functools

import jax
import jax.numpy as jnp
from jax import lax
from jax.experimental import pallas as pl
from jax.experimental.pallas import tpu as pltpu
from jax.experimental.pallas import tpu_sc as plsc

S = 128
E = 8
B = 1024
NC = 2
NS = 16
NW = NC * NS
TOK_W = B // NW
L = 16
NCH = S // L

OFF_W = 4 * S
OFF_WI = OFF_W + 2 * E * S
P_TOT = OFF_WI + 2 * E * S


def _prologue_body(m2_ref, t2_ref, p_ref):
    dot = functools.partial(
        lax.dot_general, preferred_element_type=jnp.float32,
        precision=lax.Precision.HIGHEST)
    dn_row = (((1,), (0,)), ((), ()))

    r2 = lax.broadcasted_iota(jnp.int32, (2 * S, S), 0)
    c2 = lax.broadcasted_iota(jnp.int32, (2 * S, S), 1)
    sel_r = (r2 == 2 * c2).astype(jnp.float32)
    sel_i = (r2 == 2 * c2 + 1).astype(jnp.float32)

    m2 = m2_ref[:, :]
    t2 = t2_ref[:, :]
    mr = dot(m2, sel_r, dimension_numbers=dn_row)
    mi = dot(m2, sel_i, dimension_numbers=dn_row)
    tr = dot(t2, sel_r, dimension_numbers=dn_row)
    ti = dot(t2, sel_i, dimension_numbers=dn_row)

    ex = jnp.exp(mr)
    cc = jnp.cos(mi)
    sn = jnp.sin(mi)
    zr = jnp.sum(ex * cc)
    zi = jnp.sum(ex * sn)

    ar = ex[0:1, :] * cc[0:1, :]
    ai = ex[0:1, :] * sn[0:1, :]
    br = ex[:, 0:1] * cc[:, 0:1]
    bi = ex[:, 0:1] * sn[:, 0:1]

    jj = lax.broadcasted_iota(jnp.int32, (S, S), 0)
    ss = lax.broadcasted_iota(jnp.int32, (S, S), 1)
    ang = ((jj * ss) % S).astype(jnp.float32) * (2.0 * jnp.pi / S)
    cw = jnp.cos(ang)
    sw = jnp.sin(ang)

    dn_col = (((0,), (0,)), ((), ()))
    vzr = dot(ar, cw, dimension_numbers=dn_row) + dot(
        ai, sw, dimension_numbers=dn_row)
    vzi = dot(ai, cw, dimension_numbers=dn_row) - dot(
        ar, sw, dimension_numbers=dn_row)
    uzr = dot(br, cw, dimension_numbers=dn_col) + dot(
        bi, sw, dimension_numbers=dn_col)
    uzi = dot(bi, cw, dimension_numbers=dn_col) - dot(
        br, sw, dimension_numbers=dn_col)

    den = zr * zr + zi * zi
    vr = (vzr * zr + vzi * zi) / den
    vi = (vzi * zr - vzr * zi) / den
    ur = (uzr * zr + uzi * zi) / den
    ui = (uzi * zr - uzr * zi) / den

    p_ref[0:4, :] = jnp.concatenate([vr, vi, ur, ui], axis=0)
    p_ref[4:4 + 2 * E, :] = 0.5 * jax.nn.sigmoid(tr)
    p_ref[4 + 2 * E:4 + 4 * E, :] = 0.5 * jax.nn.sigmoid(ti)


def _prologue(m2, t2):
    return pl.pallas_call(
        _prologue_body,
        out_shape=jax.ShapeDtypeStruct((4 + 4 * E, S), jnp.float32),
    )(m2, t2)


def _tree16(g):
    while len(g) > 1:
        g = [g[i] + g[i + 1] for i in range(0, len(g), 2)]
    return g[0]


def _sc_body(a_hbm, b_hbm, p_hbm, out_hbm, a_v, b_v, p_v, o_v, red_v, sum_v,
             e_v, sem):
    wid = lax.axis_index("c") * NS + lax.axis_index("s")
    base = wid * (TOK_W * S)
    cp_a = pltpu.async_copy(a_hbm.at[pl.ds(base, TOK_W * S)], a_v, sem)
    cp_b = pltpu.async_copy(b_hbm.at[pl.ds(base, TOK_W * S)], b_v, sem)
    cp_p = pltpu.async_copy(p_hbm, p_v, sem)
    cp_a.wait()
    cp_b.wait()
    cp_p.wait()

    lanes = lax.iota(jnp.int32, L)
    base16 = lanes * L
    cbase = OFF_W + (lanes & 7) * (2 * S)
    car = plsc.load_gather(p_v, [cbase])
    cbr = plsc.load_gather(p_v, [cbase + S])
    cai = plsc.load_gather(p_v, [cbase + 2 * E * S])
    cbi = plsc.load_gather(p_v, [cbase + 2 * E * S + S])

    def splat(ref, j):
        return plsc.load_gather(ref, [jnp.full((L,), j, jnp.int32)])

    def tok(t, carry):
        off = t * S
        acc_sa = jnp.zeros((L,), jnp.float32)
        acc_sb = jnp.zeros((L,), jnp.float32)
        acc_par = jnp.zeros((L,), jnp.float32)
        acc_pai = jnp.zeros((L,), jnp.float32)
        acc_pbr = jnp.zeros((L,), jnp.float32)
        acc_pbi = jnp.zeros((L,), jnp.float32)
        for c in range(NCH):
            ac = a_v[pl.ds(off + c * L, L)]
            bc = b_v[pl.ds(off + c * L, L)]
            vrc = p_v[pl.ds(0 * S + c * L, L)]
            vic = p_v[pl.ds(1 * S + c * L, L)]
            urc = p_v[pl.ds(2 * S + c * L, L)]
            uic = p_v[pl.ds(3 * S + c * L, L)]
            acc_sa = acc_sa + ac
            acc_sb = acc_sb + bc
            acc_par = acc_par + bc * vrc
            acc_pai = acc_pai + bc * vic
            acc_pbr = acc_pbr + ac * urc
            acc_pbi = acc_pbi + ac * uic
        red_v[pl.ds(1 * L, L)] = acc_sa
        red_v[pl.ds(2 * L, L)] = acc_sb
        red_v[pl.ds(3 * L, L)] = acc_par
        red_v[pl.ds(4 * L, L)] = acc_pai
        red_v[pl.ds(5 * L, L)] = acc_pbr
        red_v[pl.ds(6 * L, L)] = acc_pbi
        sums = _tree16(
            [plsc.load_gather(red_v, [base16 + k]) for k in range(L)])
        sum_v[pl.ds(0, L)] = sums
        sa = splat(sum_v, 1)
        sb = splat(sum_v, 2)
        par = splat(sum_v, 3)
        pai = splat(sum_v, 4)
        pbr = splat(sum_v, 5)
        pbi = splat(sum_v, 6)
        zar = sa * par
        zai = sa * pai
        zbr = sb * pbr
        zbi = sb * pbi
        re = zar * car + zai * cai + zbr * cbr + zbi * cbi
        im = zai * car - zar * cai + zbi * cbr - zbr * cbi
        sc = re * re + im * im
        sc = jnp.where(lanes < E, sc, -1.0)
        _, order = plsc.sort_key_val(sc, lanes, descending=True)
        e_v[pl.ds(0, L)] = order
        e_v[pl.ds(L, L)] = order
        e1 = splat(e_v, L)
        e2 = splat(e_v, 1)
        r1 = OFF_W + e1 * (2 * S)
        r2 = OFF_W + e2 * (2 * S)
        for c in range(NCH):
            col = c * L + lanes
            wa = plsc.load_gather(p_v, [r1 + col]) + plsc.load_gather(
                p_v, [r2 + col])
            wb = plsc.load_gather(p_v, [r1 + S + col]) + plsc.load_gather(
                p_v, [r2 + S + col])
            ac = a_v[pl.ds(off + c * L, L)]
            bc = b_v[pl.ds(off + c * L, L)]
            o_v[pl.ds(off + c * L, L)] = wa * ac + wb * bc
        return carry

    lax.fori_loop(0, TOK_W, tok, jnp.int32(0))
    pltpu.sync_copy(o_v, out_hbm.at[pl.ds(base, TOK_W * S)])


@functools.cache
def _sc_call():
    return pl.kernel(
        _sc_body,
        compiler_params=pltpu.CompilerParams(needs_layout_passes=False),
        out_type=jax.ShapeDtypeStruct((B * S,), jnp.float32),
        mesh=plsc.VectorSubcoreMesh(
            core_axis_name="c", subcore_axis_name="s", num_cores=NC,
            num_subcores=NS),
        scratch_types=[
            pltpu.VMEM((TOK_W * S,), jnp.float32),
            pltpu.VMEM((TOK_W * S,), jnp.float32),
            pltpu.VMEM((P_TOT,), jnp.float32),
            pltpu.VMEM((TOK_W * S,), jnp.float32),
            pltpu.VMEM((L * L,), jnp.float32),
            pltpu.VMEM((L,), jnp.float32),
            pltpu.VMEM((2 * L,), jnp.int32),
            pltpu.SemaphoreType.DMA,
        ],
    )


def kernel(a, b, mask_ri, tokens_ri):
    m2 = mask_ri.reshape(S, 2 * S)
    t2 = tokens_ri.reshape(2 * E, 2 * S)
    params = _prologue(m2, t2).reshape(-1)
    out = _sc_call()(a.reshape(B * S), b.reshape(B * S), params)
    return out.reshape(B, 1, S)


# speedup vs baseline: 1.1089x; 1.1089x over previous
import functools

import jax
import jax.numpy as jnp
from jax import lax
from jax.experimental import pallas as pl
from jax.experimental.pallas import tpu as pltpu
from jax.experimental.pallas import tpu_sc as plsc

S = 128
E = 8
B = 1024
NC = 2
NS = 16
NW = NC * NS
TOK_W = B // NW
L = 16
NCH = S // L

OFF_W = 4 * S
OFF_WI = OFF_W + 2 * E * S
P_TOT = OFF_WI + 2 * E * S


def _prologue_body(mr_ref, mi_ref, tr_ref, ti_ref, p_ref):
    dot = functools.partial(
        lax.dot_general, preferred_element_type=jnp.float32,
        precision=lax.Precision.HIGHEST)
    dn_row = (((1,), (0,)), ((), ()))

    mr = mr_ref[:, :]
    mi = mi_ref[:, :]
    tr = tr_ref[:, :]
    ti = ti_ref[:, :]

    ex = jnp.exp(mr)
    cc = jnp.cos(mi)
    sn = jnp.sin(mi)
    zr = jnp.sum(ex * cc)
    zi = jnp.sum(ex * sn)

    ar = ex[0:1, :] * cc[0:1, :]
    ai = ex[0:1, :] * sn[0:1, :]
    br = ex[:, 0:1] * cc[:, 0:1]
    bi = ex[:, 0:1] * sn[:, 0:1]

    jj = lax.broadcasted_iota(jnp.int32, (S, S), 0)
    ss = lax.broadcasted_iota(jnp.int32, (S, S), 1)
    ang = ((jj * ss) % S).astype(jnp.float32) * (2.0 * jnp.pi / S)
    cw = jnp.cos(ang)
    sw = jnp.sin(ang)

    dn_col = (((0,), (0,)), ((), ()))
    vzr = dot(ar, cw, dimension_numbers=dn_row) + dot(
        ai, sw, dimension_numbers=dn_row)
    vzi = dot(ai, cw, dimension_numbers=dn_row) - dot(
        ar, sw, dimension_numbers=dn_row)
    uzr = dot(br, cw, dimension_numbers=dn_col) + dot(
        bi, sw, dimension_numbers=dn_col)
    uzi = dot(bi, cw, dimension_numbers=dn_col) - dot(
        br, sw, dimension_numbers=dn_col)

    den = zr * zr + zi * zi
    vr = (vzr * zr + vzi * zi) / den
    vi = (vzi * zr - vzr * zi) / den
    ur = (uzr * zr + uzi * zi) / den
    ui = (uzi * zr - uzr * zi) / den

    p_ref[0:4, :] = jnp.concatenate([vr, vi, ur, ui], axis=0)
    p_ref[4:4 + 2 * E, :] = 0.5 * jax.nn.sigmoid(tr)
    p_ref[4 + 2 * E:4 + 4 * E, :] = 0.5 * jax.nn.sigmoid(ti)


def _prologue(m_r, m_i, t_r, t_i):
    return pl.pallas_call(
        _prologue_body,
        out_shape=jax.ShapeDtypeStruct((4 + 4 * E, S), jnp.float32),
    )(m_r, m_i, t_r, t_i)


def _tree16(g):
    while len(g) > 1:
        g = [g[i] + g[i + 1] for i in range(0, len(g), 2)]
    return g[0]


def _sc_body(a_hbm, b_hbm, p_hbm, out_hbm, a_v, b_v, p_v, o_v, red_v, sum_v,
             e_v, sem):
    wid = lax.axis_index("c") * NS + lax.axis_index("s")
    base = wid * (TOK_W * S)
    cp_a = pltpu.async_copy(a_hbm.at[pl.ds(base, TOK_W * S)], a_v, sem)
    cp_b = pltpu.async_copy(b_hbm.at[pl.ds(base, TOK_W * S)], b_v, sem)
    cp_p = pltpu.async_copy(p_hbm, p_v, sem)
    cp_a.wait()
    cp_b.wait()
    cp_p.wait()

    lanes = lax.iota(jnp.int32, L)
    base16 = lanes * L
    cbase = OFF_W + (lanes & 7) * (2 * S)
    car = plsc.load_gather(p_v, [cbase])
    cbr = plsc.load_gather(p_v, [cbase + S])
    cai = plsc.load_gather(p_v, [cbase + 2 * E * S])
    cbi = plsc.load_gather(p_v, [cbase + 2 * E * S + S])

    def splat(ref, j):
        return plsc.load_gather(ref, [jnp.full((L,), j, jnp.int32)])

    def tok(t, carry):
        offa = (2 * t) * S
        offb = offa + S
        acc = [jnp.zeros((L,), jnp.float32) for _ in range(12)]
        for c in range(NCH):
            aca = a_v[pl.ds(offa + c * L, L)]
            bca = b_v[pl.ds(offa + c * L, L)]
            acb = a_v[pl.ds(offb + c * L, L)]
            bcb = b_v[pl.ds(offb + c * L, L)]
            vrc = p_v[pl.ds(0 * S + c * L, L)]
            vic = p_v[pl.ds(1 * S + c * L, L)]
            urc = p_v[pl.ds(2 * S + c * L, L)]
            uic = p_v[pl.ds(3 * S + c * L, L)]
            acc[0] = acc[0] + aca
            acc[1] = acc[1] + bca
            acc[2] = acc[2] + bca * vrc
            acc[3] = acc[3] + bca * vic
            acc[4] = acc[4] + aca * urc
            acc[5] = acc[5] + aca * uic
            acc[6] = acc[6] + acb
            acc[7] = acc[7] + bcb
            acc[8] = acc[8] + bcb * vrc
            acc[9] = acc[9] + bcb * vic
            acc[10] = acc[10] + acb * urc
            acc[11] = acc[11] + acb * uic
        for j in range(6):
            red_v[pl.ds((1 + j) * L, L)] = acc[j]
            red_v[pl.ds((9 + j) * L, L)] = acc[6 + j]
        sums = _tree16(
            [plsc.load_gather(red_v, [base16 + k]) for k in range(L)])
        sum_v[pl.ds(0, L)] = sums

        def route(sbase):
            sa = splat(sum_v, sbase + 1)
            sb = splat(sum_v, sbase + 2)
            par = splat(sum_v, sbase + 3)
            pai = splat(sum_v, sbase + 4)
            pbr = splat(sum_v, sbase + 5)
            pbi = splat(sum_v, sbase + 6)
            zar = sa * par
            zai = sa * pai
            zbr = sb * pbr
            zbi = sb * pbi
            re = zar * car + zai * cai + zbr * cbr + zbi * cbi
            im = zai * car - zar * cai + zbi * cbr - zbr * cbi
            sc = re * re + im * im
            sc = jnp.where(lanes < E, sc, -1.0)
            _, order = plsc.sort_key_val(sc, lanes, descending=True)
            return order

        orda = route(0)
        ordb = route(8)
        e_v[pl.ds(0, L)] = orda
        e_v[pl.ds(L, L)] = orda
        e_v[pl.ds(2 * L, L)] = ordb
        e_v[pl.ds(3 * L, L)] = ordb
        r1a = OFF_W + splat(e_v, L) * (2 * S)
        r2a = OFF_W + splat(e_v, 1) * (2 * S)
        r1b = OFF_W + splat(e_v, 3 * L) * (2 * S)
        r2b = OFF_W + splat(e_v, 2 * L + 1) * (2 * S)
        for c in range(NCH):
            col = c * L + lanes
            waa = plsc.load_gather(p_v, [r1a + col]) + plsc.load_gather(
                p_v, [r2a + col])
            wba = plsc.load_gather(p_v, [r1a + S + col]) + plsc.load_gather(
                p_v, [r2a + S + col])
            wab = plsc.load_gather(p_v, [r1b + col]) + plsc.load_gather(
                p_v, [r2b + col])
            wbb = plsc.load_gather(p_v, [r1b + S + col]) + plsc.load_gather(
                p_v, [r2b + S + col])
            aca = a_v[pl.ds(offa + c * L, L)]
            bca = b_v[pl.ds(offa + c * L, L)]
            acb = a_v[pl.ds(offb + c * L, L)]
            bcb = b_v[pl.ds(offb + c * L, L)]
            o_v[pl.ds(offa + c * L, L)] = waa * aca + wba * bca
            o_v[pl.ds(offb + c * L, L)] = wab * acb + wbb * bcb
        return carry

    lax.fori_loop(0, TOK_W // 2, tok, jnp.int32(0))
    pltpu.sync_copy(o_v, out_hbm.at[pl.ds(base, TOK_W * S)])


@functools.cache
def _sc_call():
    return pl.kernel(
        _sc_body,
        compiler_params=pltpu.CompilerParams(needs_layout_passes=False),
        out_type=jax.ShapeDtypeStruct((B * S,), jnp.float32),
        mesh=plsc.VectorSubcoreMesh(
            core_axis_name="c", subcore_axis_name="s", num_cores=NC,
            num_subcores=NS),
        scratch_types=[
            pltpu.VMEM((TOK_W * S,), jnp.float32),
            pltpu.VMEM((TOK_W * S,), jnp.float32),
            pltpu.VMEM((P_TOT,), jnp.float32),
            pltpu.VMEM((TOK_W * S,), jnp.float32),
            pltpu.VMEM((L * L,), jnp.float32),
            pltpu.VMEM((L,), jnp.float32),
            pltpu.VMEM((4 * L,), jnp.int32),
            pltpu.SemaphoreType.DMA,
        ],
    )


def kernel(a, b, mask_ri, tokens_ri):
    m_r = mask_ri[..., 0]
    m_i = mask_ri[..., 1]
    t_r = tokens_ri[..., 0].reshape(2 * E, S)
    t_i = tokens_ri[..., 1].reshape(2 * E, S)
    params = _prologue(m_r, m_i, t_r, t_i).reshape(-1)
    out = _sc_call()(a.reshape(B * S), b.reshape(B * S), params)
    return out.reshape(B, 1, S)
